# baseline (speedup 1.0000x reference)
import jax
import jax.numpy as jnp
from jax.experimental import pallas as pl
from jax.experimental.pallas import tpu as pltpu

_M = 100
_N = 100
_S = _M * _N
_BR = 2048
_BC = 2048


def _adj_block_kernel(thr_ref, dc_ref, dm101_ref, dm99_ref, dp99_ref,
                      dp101_ref, out_ref):
    rb = pl.program_id(0)
    cb = pl.program_id(1)
    r0 = rb * _BR
    c0 = cb * _BC
    on_band = (c0 <= r0 + (_BR - 1) + 101) & (c0 + (_BC - 1) >= r0 - 101)

    @pl.when(jnp.logical_not(on_band))
    def _():
        out_ref[...] = jnp.zeros((_BR, _BC), jnp.bool_)

    @pl.when(on_band)
    def _():
        t = thr_ref[0, 0]
        dc = dc_ref[...]
        dm101 = dm101_ref[...]
        dm99 = dm99_ref[...]
        dp99 = dp99_ref[...]
        dp101 = dp101_ref[...]

        c = jax.lax.broadcasted_iota(jnp.int32, (1, _BC), 1) + c0
        xa = c - 101
        a_ok = (xa >= 0) & (xa < 9900) & (xa % 100 < 99)
        a = a_ok & (jnp.abs(dc - dm101) <= t)
        xb = c - 99
        b_ok = (xb >= 0) & (xb < 9900) & (xb % 100 > 0)
        b = b_ok & (jnp.abs(dc - dm99) <= t)
        c_ok = (c < 9900) & (c % 100 > 0)
        cm = c_ok & (jnp.abs(dp99 - dc) <= t)
        d_ok = (c < 9900) & (c % 100 < 99)
        dm = d_ok & (jnp.abs(dp101 - dc) <= t)

        rows = jax.lax.broadcasted_iota(jnp.int32, (_BR, _BC), 0) + r0
        delta = c - rows
        out = (((delta == 101) & a) | ((delta == 99) & b)
               | ((delta == -99) & cm) | ((delta == -101) & dm))
        out_ref[...] = out


def kernel(d_noised, threshold):
    dflat = d_noised.reshape(1, _S)
    padded = jnp.pad(dflat, ((0, 0), (101, 101)))
    dm101 = padded[:, 0:_S]
    dm99 = padded[:, 2:2 + _S]
    dp99 = padded[:, 200:200 + _S]
    dp101 = padded[:, 202:202 + _S]
    thr = threshold.reshape(1, 1)

    nrb = pl.cdiv(_S, _BR)
    ncb = pl.cdiv(_S, _BC)
    col_spec = pl.BlockSpec((1, _BC), lambda rb, cb: (0, cb))
    return pl.pallas_call(
        _adj_block_kernel,
        grid=(nrb, ncb),
        in_specs=[
            pl.BlockSpec(memory_space=pltpu.SMEM),
            col_spec, col_spec, col_spec, col_spec, col_spec,
        ],
        out_specs=pl.BlockSpec((_BR, _BC), lambda rb, cb: (rb, cb)),
        out_shape=jax.ShapeDtypeStruct((_S, _S), jnp.bool_),
    )(thr, dflat, dm101, dm99, dp99, dp101)


# speedup vs baseline: 1.1762x; 1.1762x over previous
import jax
import jax.numpy as jnp
from jax.experimental import pallas as pl
from jax.experimental.pallas import tpu as pltpu

_M = 100
_N = 100
_S = _M * _N
_BR = 512
_BC = 1024
_NRB = pl.cdiv(_S, _BR)
_NCB = pl.cdiv(_S, _BC)


def _win0(rb):
    return jnp.clip((_BR * rb - 101) // _BC, 0, _NCB - 2)


def _band_block_kernel(thr_ref, dc_ref, dm101_ref, dm99_ref, dp99_ref,
                       dp101_ref, zeros_ref, out_ref):
    del zeros_ref
    rb = pl.program_id(0)
    cb = pl.program_id(1)
    r0 = rb * _BR
    c0 = (_win0(rb) + cb) * _BC

    t = thr_ref[0, 0]
    dc = dc_ref[...]
    dm101 = dm101_ref[...]
    dm99 = dm99_ref[...]
    dp99 = dp99_ref[...]
    dp101 = dp101_ref[...]

    c = jax.lax.broadcasted_iota(jnp.int32, (1, _BC), 1) + c0
    xa = c - 101
    a_ok = (xa >= 0) & (xa < 9900) & (xa % 100 < 99)
    a = a_ok & (jnp.abs(dc - dm101) <= t)
    xb = c - 99
    b_ok = (xb >= 0) & (xb < 9900) & (xb % 100 > 0)
    b = b_ok & (jnp.abs(dc - dm99) <= t)
    c_ok = (c < 9900) & (c % 100 > 0)
    cm = c_ok & (jnp.abs(dp99 - dc) <= t)
    d_ok = (c < 9900) & (c % 100 < 99)
    dm = d_ok & (jnp.abs(dp101 - dc) <= t)

    rows = jax.lax.broadcasted_iota(jnp.int32, (_BR, _BC), 0) + r0
    delta = c - rows
    out = (((delta == 101) & a) | ((delta == 99) & b)
           | ((delta == -99) & cm) | ((delta == -101) & dm))
    out_ref[...] = out


def kernel(d_noised, threshold):
    dflat = d_noised.reshape(1, _S)
    padded = jnp.pad(dflat, ((0, 0), (101, 101)))
    dm101 = padded[:, 0:_S]
    dm99 = padded[:, 2:2 + _S]
    dp99 = padded[:, 200:200 + _S]
    dp101 = padded[:, 202:202 + _S]
    thr = threshold.reshape(1, 1)
    zeros = jnp.zeros((_S, _S), jnp.bool_)

    col_spec = pl.BlockSpec((1, _BC), lambda rb, cb: (0, _win0(rb) + cb))
    return pl.pallas_call(
        _band_block_kernel,
        grid=(_NRB, 2),
        in_specs=[
            pl.BlockSpec(memory_space=pltpu.SMEM),
            col_spec, col_spec, col_spec, col_spec, col_spec,
            pl.BlockSpec(memory_space=pl.ANY),
        ],
        out_specs=pl.BlockSpec((_BR, _BC),
                               lambda rb, cb: (rb, _win0(rb) + cb)),
        out_shape=jax.ShapeDtypeStruct((_S, _S), jnp.bool_),
        input_output_aliases={6: 0},
    )(thr, dflat, dm101, dm99, dp99, dp101, zeros)


# speedup vs baseline: 1.5989x; 1.3594x over previous
import jax
import jax.numpy as jnp
from jax.experimental import pallas as pl
from jax.experimental.pallas import tpu as pltpu

_M = 100
_N = 100
_S = _M * _N
_BR = 1024
_BC = 1024


def _adj_block_kernel(thr_ref, dc_ref, dm101_ref, dm99_ref, dp99_ref,
                      dp101_ref, out_ref):
    rb = pl.program_id(0)
    cb = pl.program_id(1)
    r0 = rb * _BR
    c0 = cb * _BC
    on_band = (c0 <= r0 + (_BR - 1) + 101) & (c0 + (_BC - 1) >= r0 - 101)

    @pl.when(jnp.logical_not(on_band))
    def _():
        out_ref[...] = jnp.zeros((_BR, _BC), jnp.int8)

    @pl.when(on_band)
    def _():
        t = thr_ref[0, 0]
        dc = dc_ref[...]
        dm101 = dm101_ref[...]
        dm99 = dm99_ref[...]
        dp99 = dp99_ref[...]
        dp101 = dp101_ref[...]

        c = jax.lax.broadcasted_iota(jnp.int32, (1, _BC), 1) + c0
        xa = c - 101
        a_ok = (xa >= 0) & (xa < 9900) & (xa % 100 < 99)
        a = a_ok & (jnp.abs(dc - dm101) <= t)
        xb = c - 99
        b_ok = (xb >= 0) & (xb < 9900) & (xb % 100 > 0)
        b = b_ok & (jnp.abs(dc - dm99) <= t)
        c_ok = (c < 9900) & (c % 100 > 0)
        cm = c_ok & (jnp.abs(dp99 - dc) <= t)
        d_ok = (c < 9900) & (c % 100 < 99)
        dm = d_ok & (jnp.abs(dp101 - dc) <= t)

        rows = jax.lax.broadcasted_iota(jnp.int32, (_BR, _BC), 0) + r0
        delta = c - rows
        out = (((delta == 101) & a) | ((delta == 99) & b)
               | ((delta == -99) & cm) | ((delta == -101) & dm))
        out_ref[...] = out.astype(jnp.int8)


def kernel(d_noised, threshold):
    dflat = d_noised.reshape(1, _S)
    padded = jnp.pad(dflat, ((0, 0), (101, 101)))
    dm101 = padded[:, 0:_S]
    dm99 = padded[:, 2:2 + _S]
    dp99 = padded[:, 200:200 + _S]
    dp101 = padded[:, 202:202 + _S]
    thr = threshold.reshape(1, 1)

    nrb = pl.cdiv(_S, _BR)
    ncb = pl.cdiv(_S, _BC)
    col_spec = pl.BlockSpec((1, _BC), lambda rb, cb: (0, cb))
    return pl.pallas_call(
        _adj_block_kernel,
        grid=(nrb, ncb),
        in_specs=[
            pl.BlockSpec(memory_space=pltpu.SMEM),
            col_spec, col_spec, col_spec, col_spec, col_spec,
        ],
        out_specs=pl.BlockSpec((_BR, _BC), lambda rb, cb: (rb, cb)),
        out_shape=jax.ShapeDtypeStruct((_S, _S), jnp.int8),
    )(thr, dflat, dm101, dm99, dp99, dp101).astype(jnp.bool_)


# speedup vs baseline: 2.4142x; 1.5100x over previous
import jax
import jax.numpy as jnp
from jax.experimental import pallas as pl
from jax.experimental.pallas import tpu as pltpu

_M = 100
_N = 100
_S = _M * _N
_BR = 256
_BC = 512
_NRB = pl.cdiv(_S, _BR)
_NCB = pl.cdiv(_S, _BC)


def _win0(rb):
    return jnp.clip((_BR * rb - 101) // _BC, 0, _NCB - 2)


def _band_block_kernel(thr_ref, dc_ref, dm101_ref, dm99_ref, dp99_ref,
                       dp101_ref, zeros_ref, out_ref):
    del zeros_ref
    rb = pl.program_id(0)
    cb = pl.program_id(1)
    r0 = rb * _BR
    c0 = (_win0(rb) + cb) * _BC

    t = thr_ref[0, 0]
    dc = dc_ref[...]
    dm101 = dm101_ref[...]
    dm99 = dm99_ref[...]
    dp99 = dp99_ref[...]
    dp101 = dp101_ref[...]

    c = jax.lax.broadcasted_iota(jnp.int32, (1, _BC), 1) + c0
    xa = c - 101
    a_ok = (xa >= 0) & (xa < 9900) & (xa % 100 < 99)
    a = a_ok & (jnp.abs(dc - dm101) <= t)
    xb = c - 99
    b_ok = (xb >= 0) & (xb < 9900) & (xb % 100 > 0)
    b = b_ok & (jnp.abs(dc - dm99) <= t)
    c_ok = (c < 9900) & (c % 100 > 0)
    cm = c_ok & (jnp.abs(dp99 - dc) <= t)
    d_ok = (c < 9900) & (c % 100 < 99)
    dm = d_ok & (jnp.abs(dp101 - dc) <= t)

    rows = jax.lax.broadcasted_iota(jnp.int32, (_BR, _BC), 0) + r0
    delta = c - rows
    out = (((delta == 101) & a) | ((delta == 99) & b)
           | ((delta == -99) & cm) | ((delta == -101) & dm))
    out_ref[...] = out.astype(jnp.int8)


def kernel(d_noised, threshold):
    dflat = d_noised.reshape(1, _S)
    padded = jnp.pad(dflat, ((0, 0), (101, 101)))
    dm101 = padded[:, 0:_S]
    dm99 = padded[:, 2:2 + _S]
    dp99 = padded[:, 200:200 + _S]
    dp101 = padded[:, 202:202 + _S]
    thr = threshold.reshape(1, 1)
    zeros = jnp.zeros((_S, _S), jnp.int8)

    col_spec = pl.BlockSpec((1, _BC), lambda rb, cb: (0, _win0(rb) + cb))
    out = pl.pallas_call(
        _band_block_kernel,
        grid=(_NRB, 2),
        in_specs=[
            pl.BlockSpec(memory_space=pltpu.SMEM),
            col_spec, col_spec, col_spec, col_spec, col_spec,
            pl.BlockSpec(memory_space=pl.ANY),
        ],
        out_specs=pl.BlockSpec((_BR, _BC),
                               lambda rb, cb: (rb, _win0(rb) + cb)),
        out_shape=jax.ShapeDtypeStruct((_S, _S), jnp.int8),
        input_output_aliases={6: 0},
    )(thr, dflat, dm101, dm99, dp99, dp101, zeros)
    return out.astype(jnp.bool_)
